# baseline (speedup 1.0000x reference)
import functools

import jax
import jax.numpy as jnp
from jax import lax
from jax.experimental import pallas as pl
from jax.experimental.pallas import tpu as pltpu
from jax.experimental.pallas import tpu_sc as plsc

NC, NS = 2, 16
NW = NC * NS
LANES = 16
KPAD = 102400
SLICE = KPAD // NW
BLK = 6400
CPT = LANES
NCAND = NW * CPT
TOPK = 6
PAD_D2 = 1e30


def _dist_body(nkeys, q_ref, k_ref, o_ref):
    i = pl.program_id(0)
    q = q_ref[...]
    kb = k_ref[...]
    qk = lax.dot_general(q, kb, (((1,), (1,)), ((), ())),
                         preferred_element_type=jnp.float32,
                         precision=lax.Precision.HIGHEST)
    ones = jnp.ones((1, kb.shape[1]), jnp.float32)
    kk = lax.dot_general(ones, kb * kb, (((1,), (1,)), ((), ())),
                         preferred_element_type=jnp.float32,
                         precision=lax.Precision.HIGHEST)
    qq = jnp.sum(q * q, axis=1, keepdims=True)
    d2 = qq + kk - 2.0 * qk
    gid = i * BLK + lax.broadcasted_iota(jnp.int32, d2.shape, 1)
    o_ref[...] = jnp.where(gid >= nkeys, jnp.full_like(d2, PAD_D2), d2)


def _sc_topk_body(nq, d2_hbm, keys_hbm, rows_out, idx_out,
                  d2_v, idx_v, rows_v, sem):
    wid = lax.axis_index("s") * NC + lax.axis_index("c")
    base = wid * SLICE
    pltpu.sync_copy(d2_hbm.at[:, pl.ds(base, SLICE)], d2_v)
    lanes = lax.iota(jnp.int32, LANES)
    inf_v = jnp.full((LANES,), jnp.inf, jnp.float32)
    zero_i = jnp.zeros((LANES,), jnp.int32)

    def step(i, carry):
        off = base + i * CPT
        out = []
        for r in range(nq):
            bv, bi = carry[r]
            v = d2_v[r, pl.ds(i * CPT, CPT)]
            iv = lanes + off
            vd, ivd = plsc.sort_key_val(v, iv, descending=True)
            take = vd < bv
            mv = jnp.where(take, vd, bv)
            mi = jnp.where(take, ivd, bi)
            out.append(tuple(plsc.sort_key_val(mv, mi)))
        return tuple(out)

    init = tuple((inf_v, zero_i) for _ in range(nq))
    res = lax.fori_loop(0, SLICE // CPT, step, init)
    for r in range(nq):
        bv, bi = res[r]
        idx_v[...] = bi
        pltpu.async_copy(keys_hbm.at[bi], rows_v, sem).wait()
        pltpu.sync_copy(rows_v, rows_out.at[r, pl.ds(wid * CPT, CPT)])
        pltpu.sync_copy(idx_v, idx_out.at[r, pl.ds(wid * CPT, CPT)])


def _final_body(q_ref, rows_ref, ci_ref, vals_ref, idx_ref):
    q = q_ref[...]
    rows = rows_ref[...]
    diff = q[:, None, :] - rows
    d2 = jnp.sum(diff * diff, axis=2)
    dist = jnp.sqrt(d2)
    im = ci_ref[...]
    intmax = jnp.int32(2147483647)
    big = jnp.float32(3e38)
    work = dist
    vs, ids = [], []
    for _ in range(TOPK):
        m = jnp.min(work, axis=1, keepdims=True)
        tie = jnp.where(work == m, im, intmax)
        ci = jnp.min(tie, axis=1, keepdims=True)
        vs.append(m)
        ids.append(ci)
        work = jnp.where(im == ci, big, work)
    vals_ref[...] = jnp.concatenate(vs, axis=1)
    idx_ref[...] = jnp.concatenate(ids, axis=1)


def kernel(queries, keys, k):
    nq, d = queries.shape
    nkeys = keys.shape[0]

    d2 = pl.pallas_call(
        functools.partial(_dist_body, nkeys),
        grid=(KPAD // BLK,),
        in_specs=[
            pl.BlockSpec((nq, d), lambda i: (0, 0)),
            pl.BlockSpec((BLK, d), lambda i: (i, 0)),
        ],
        out_specs=pl.BlockSpec((nq, BLK), lambda i: (0, i)),
        out_shape=jax.ShapeDtypeStruct((nq, KPAD), jnp.float32),
    )(queries, keys)

    mesh = plsc.VectorSubcoreMesh(core_axis_name="c", subcore_axis_name="s",
                                  num_cores=NC, num_subcores=NS)
    sc_topk = functools.partial(
        pl.kernel,
        out_type=(jax.ShapeDtypeStruct((nq, NCAND, d), jnp.float32),
                  jax.ShapeDtypeStruct((nq, NCAND), jnp.int32)),
        mesh=mesh,
        compiler_params=pltpu.CompilerParams(needs_layout_passes=False,
                                             use_tc_tiling_on_sc=False),
        scratch_types=[
            pltpu.VMEM((nq, SLICE), jnp.float32),
            pltpu.VMEM((CPT,), jnp.int32),
            pltpu.VMEM((CPT, d), jnp.float32),
            pltpu.SemaphoreType.DMA,
        ],
    )(functools.partial(_sc_topk_body, nq))
    rows, cidx = sc_topk(d2, keys)

    vals, idx = pl.pallas_call(
        _final_body,
        in_specs=[
            pl.BlockSpec(queries.shape, lambda: (0, 0)),
            pl.BlockSpec((nq, NCAND, d), lambda: (0, 0, 0)),
            pl.BlockSpec((nq, NCAND), lambda: (0, 0)),
        ],
        out_specs=[
            pl.BlockSpec((nq, TOPK), lambda: (0, 0)),
            pl.BlockSpec((nq, TOPK), lambda: (0, 0)),
        ],
        out_shape=[
            jax.ShapeDtypeStruct((nq, TOPK), jnp.float32),
            jax.ShapeDtypeStruct((nq, TOPK), jnp.int32),
        ],
    )(queries, rows, cidx)
    return vals, idx


# speedup vs baseline: 1.9700x; 1.9700x over previous
import functools

import jax
import jax.numpy as jnp
from jax import lax
from jax.experimental import pallas as pl
from jax.experimental.pallas import tpu as pltpu
from jax.experimental.pallas import tpu_sc as plsc

NC, NS = 2, 16
NW = NC * NS
LANES = 16
KPAD = 102400
SLICE = KPAD // NW
TPW = SLICE // 128
BLK = 6400
CPT = LANES
NCAND = NW * CPT
TOPK = 6
PAD_D2 = 1e30


def _dist_body(nkeys, q_ref, kt_ref, o_ref, k2_ref):
    i = pl.program_id(0)
    q = q_ref[...]
    kb = kt_ref[...]
    qk = lax.dot_general(q, kb, (((1,), (0,)), ((), ())),
                         preferred_element_type=jnp.float32,
                         precision=lax.Precision.HIGHEST)
    ones = jnp.ones((1, kb.shape[0]), jnp.float32)
    kk = lax.dot_general(ones, kb * kb, (((1,), (0,)), ((), ())),
                         preferred_element_type=jnp.float32,
                         precision=lax.Precision.HIGHEST)
    qq = jnp.sum(q * q, axis=1, keepdims=True)
    d2 = qq + kk - 2.0 * qk
    gid = i * BLK + lax.broadcasted_iota(jnp.int32, d2.shape, 1)
    d2 = jnp.where(gid >= nkeys, jnp.full_like(d2, PAD_D2), d2)
    nq = d2.shape[0]
    o_ref[...] = jnp.transpose(d2.reshape(nq, BLK // 128, 128), (1, 0, 2))
    dd = kb.shape[0]
    ii = lax.broadcasted_iota(jnp.int32, (dd, dd), 0)
    jj = lax.broadcasted_iota(jnp.int32, (dd, dd), 1)
    eye = (ii == jj).astype(jnp.float32)
    kbt = lax.dot_general(kb, eye, (((0,), (0,)), ((), ())),
                          preferred_element_type=jnp.float32,
                          precision=lax.Precision.HIGHEST)
    k2_ref[:, pl.ds(0, dd)] = kbt


def _sc_topk_body(nq, d2_hbm, keys2_hbm, rows_out, idx_out,
                  d2_v, idx_v, rows_v, sem):
    wid = lax.axis_index("s") * NC + lax.axis_index("c")
    base = wid * SLICE
    pltpu.sync_copy(d2_hbm.at[pl.ds(wid * TPW, TPW)], d2_v)
    lanes = lax.iota(jnp.int32, LANES)
    inf_v = jnp.full((LANES,), jnp.inf, jnp.float32)
    zero_i = jnp.zeros((LANES,), jnp.int32)

    def step(i, carry):
        off = base + i * CPT
        tt = i // 8
        lo = (i % 8) * CPT
        out = []
        for r in range(nq):
            bv, bi = carry[r]
            v = d2_v[tt, r, pl.ds(lo, CPT)]
            iv = lanes + off
            vd, ivd = plsc.sort_key_val(v, iv, descending=True)
            take = vd < bv
            mv = jnp.where(take, vd, bv)
            mi = jnp.where(take, ivd, bi)
            out.append(tuple(plsc.sort_key_val(mv, mi)))
        return tuple(out)

    init = tuple((inf_v, zero_i) for _ in range(nq))
    res = lax.fori_loop(0, SLICE // CPT, step, init)
    for r in range(nq):
        bv, bi = res[r]
        idx_v[...] = bi
        pltpu.async_copy(keys2_hbm.at[bi], rows_v, sem).wait()
        pltpu.sync_copy(rows_v, rows_out.at[pl.ds(r * NCAND + wid * CPT, CPT)])
        pltpu.sync_copy(idx_v, idx_out.at[r * (NCAND // 128) + wid // 8,
                                          pl.ds((wid % 8) * CPT, CPT)])


def _final_body(q_ref, rows_ref, ci_ref, vals_ref, idx_ref):
    q = q_ref[...]
    nq, dd = q.shape
    rows = rows_ref[...].reshape(nq, NCAND, 2 * dd)[:, :, :dd]
    diff = q[:, None, :] - rows
    sq = diff * diff
    d2 = jnp.sum(sq, axis=2)
    im = ci_ref[...].reshape(nq, NCAND)
    dist = jnp.sqrt(d2)
    intmax = jnp.int32(2147483647)
    big = jnp.float32(3e38)
    work = dist
    vs, ids = [], []
    for _ in range(TOPK):
        m = jnp.min(work, axis=1, keepdims=True)
        tie = jnp.where(work == m, im, intmax)
        ci = jnp.min(tie, axis=1, keepdims=True)
        vs.append(m)
        ids.append(ci)
        work = jnp.where(im == ci, big, work)
    vals_ref[...] = jnp.concatenate(vs, axis=1)
    idx_ref[...] = jnp.concatenate(ids, axis=1)


def kernel(queries, keys, k):
    nq, d = queries.shape
    nkeys = keys.shape[0]
    keys_t = keys.T

    d2, keys2 = pl.pallas_call(
        functools.partial(_dist_body, nkeys),
        grid=(KPAD // BLK,),
        in_specs=[
            pl.BlockSpec((nq, d), lambda i: (0, 0)),
            pl.BlockSpec((d, BLK), lambda i: (0, i)),
        ],
        out_specs=[
            pl.BlockSpec((BLK // 128, nq, 128), lambda i: (i, 0, 0)),
            pl.BlockSpec((BLK, 2 * d), lambda i: (i, 0)),
        ],
        out_shape=[
            jax.ShapeDtypeStruct((KPAD // 128, nq, 128), jnp.float32),
            jax.ShapeDtypeStruct((KPAD, 2 * d), jnp.float32),
        ],
    )(queries, keys_t)

    mesh = plsc.VectorSubcoreMesh(core_axis_name="c", subcore_axis_name="s",
                                  num_cores=NC, num_subcores=NS)
    sc_topk = functools.partial(
        pl.kernel,
        out_type=(jax.ShapeDtypeStruct((nq * NCAND, 128), jnp.float32),
                  jax.ShapeDtypeStruct((nq * NCAND // 128, 128), jnp.int32)),
        mesh=mesh,
        compiler_params=pltpu.CompilerParams(needs_layout_passes=False,
                                             use_tc_tiling_on_sc=False),
        scratch_types=[
            pltpu.VMEM((TPW, nq, 128), jnp.float32),
            pltpu.VMEM((CPT,), jnp.int32),
            pltpu.VMEM((CPT, 128), jnp.float32),
            pltpu.SemaphoreType.DMA,
        ],
    )(functools.partial(_sc_topk_body, nq))
    rows, cidx = sc_topk(d2, keys2)

    vals, idx = pl.pallas_call(
        _final_body,
        in_specs=[
            pl.BlockSpec(queries.shape, lambda: (0, 0)),
            pl.BlockSpec((nq * NCAND, 128), lambda: (0, 0)),
            pl.BlockSpec((nq * NCAND // 128, 128), lambda: (0, 0)),
        ],
        out_specs=[
            pl.BlockSpec((nq, TOPK), lambda: (0, 0)),
            pl.BlockSpec((nq, TOPK), lambda: (0, 0)),
        ],
        out_shape=[
            jax.ShapeDtypeStruct((nq, TOPK), jnp.float32),
            jax.ShapeDtypeStruct((nq, TOPK), jnp.int32),
        ],
    )(queries, rows, cidx)
    return vals, idx


# speedup vs baseline: 2.8512x; 1.4474x over previous
import functools

import jax
import jax.numpy as jnp
from jax import lax
from jax.experimental import pallas as pl
from jax.experimental.pallas import tpu as pltpu
from jax.experimental.pallas import tpu_sc as plsc

NC, NS = 2, 16
NW = NC * NS
LANES = 16
KPAD = 102400
SLICE = KPAD // NW
TPW = SLICE // 128
BLK = 6400
CPT = LANES
NCAND = NW * CPT
TOPK = 6
PAD_D2 = 1e30


def _dist_body(nkeys, q_ref, kt_ref, o_ref, k2_ref):
    i = pl.program_id(0)
    q = q_ref[...]
    kb = kt_ref[...]
    qk = lax.dot_general(q, kb, (((1,), (0,)), ((), ())),
                         preferred_element_type=jnp.float32,
                         precision=lax.Precision.HIGHEST)
    ones = jnp.ones((1, kb.shape[0]), jnp.float32)
    kk = lax.dot_general(ones, kb * kb, (((1,), (0,)), ((), ())),
                         preferred_element_type=jnp.float32,
                         precision=lax.Precision.HIGHEST)
    qq = jnp.sum(q * q, axis=1, keepdims=True)
    d2 = qq + kk - 2.0 * qk
    gid = i * BLK + lax.broadcasted_iota(jnp.int32, d2.shape, 1)
    d2 = jnp.where(gid >= nkeys, jnp.full_like(d2, PAD_D2), d2)
    nq = d2.shape[0]
    o_ref[...] = jnp.transpose(d2.reshape(nq, BLK // 128, 128), (1, 0, 2))
    dd = kb.shape[0]
    kbt = lax.transpose(kb, (1, 0))
    k2_ref[:, pl.ds(0, dd)] = kbt


def _sc_topk_body(nq, d2_hbm, keys2_hbm, rows_out, idx_out,
                  d2_v, idx_v, rows_v, sem):
    wid = lax.axis_index("s") * NC + lax.axis_index("c")
    base = wid * SLICE
    pltpu.sync_copy(d2_hbm.at[pl.ds(wid * TPW, TPW)], d2_v)
    lanes = lax.iota(jnp.int32, LANES)
    inf_v = jnp.full((LANES,), jnp.inf, jnp.float32)
    zero_i = jnp.zeros((LANES,), jnp.int32)

    def step(i, carry):
        off = base + i * CPT
        tt = i // 8
        lo = (i % 8) * CPT
        out = []
        for r in range(nq):
            bv, bi = carry[r]
            v = d2_v[tt, r, pl.ds(lo, CPT)]
            iv = lanes + off
            vd, ivd = plsc.sort_key_val(v, iv, descending=True)
            take = vd < bv
            mv = jnp.where(take, vd, bv)
            mi = jnp.where(take, ivd, bi)
            out.append(tuple(plsc.sort_key_val(mv, mi)))
        return tuple(out)

    init = tuple((inf_v, zero_i) for _ in range(nq))
    res = lax.fori_loop(0, SLICE // CPT, step, init)
    for r in range(nq):
        bv, bi = res[r]
        idx_v[...] = bi
        pltpu.async_copy(keys2_hbm.at[bi], rows_v, sem).wait()
        pltpu.sync_copy(rows_v, rows_out.at[pl.ds(r * NCAND + wid * CPT, CPT)])
        pltpu.sync_copy(idx_v, idx_out.at[r * (NCAND // 128) + wid // 8,
                                          pl.ds((wid % 8) * CPT, CPT)])


def _final_body(q_ref, rows_ref, ci_ref, vals_ref, idx_ref):
    q = q_ref[...]
    nq, dd = q.shape
    rows = rows_ref[...].reshape(nq, NCAND, 2 * dd)[:, :, :dd]
    diff = q[:, None, :] - rows
    sq = diff * diff
    d2 = jnp.sum(sq, axis=2)
    im = ci_ref[...].reshape(nq, NCAND)
    dist = jnp.sqrt(d2)
    intmax = jnp.int32(2147483647)
    big = jnp.float32(3e38)
    work = dist
    vs, ids = [], []
    for _ in range(TOPK):
        m = jnp.min(work, axis=1, keepdims=True)
        tie = jnp.where(work == m, im, intmax)
        ci = jnp.min(tie, axis=1, keepdims=True)
        vs.append(m)
        ids.append(ci)
        work = jnp.where(im == ci, big, work)
    vals_ref[...] = jnp.concatenate(vs, axis=1)
    idx_ref[...] = jnp.concatenate(ids, axis=1)


def kernel(queries, keys, k):
    nq, d = queries.shape
    nkeys = keys.shape[0]
    keys_t = keys.T

    d2, keys2 = pl.pallas_call(
        functools.partial(_dist_body, nkeys),
        grid=(KPAD // BLK,),
        in_specs=[
            pl.BlockSpec((nq, d), lambda i: (0, 0)),
            pl.BlockSpec((d, BLK), lambda i: (0, i)),
        ],
        out_specs=[
            pl.BlockSpec((BLK // 128, nq, 128), lambda i: (i, 0, 0)),
            pl.BlockSpec((BLK, 2 * d), lambda i: (i, 0)),
        ],
        out_shape=[
            jax.ShapeDtypeStruct((KPAD // 128, nq, 128), jnp.float32),
            jax.ShapeDtypeStruct((KPAD, 2 * d), jnp.float32),
        ],
    )(queries, keys_t)

    mesh = plsc.VectorSubcoreMesh(core_axis_name="c", subcore_axis_name="s",
                                  num_cores=NC, num_subcores=NS)
    sc_topk = functools.partial(
        pl.kernel,
        out_type=(jax.ShapeDtypeStruct((nq * NCAND, 128), jnp.float32),
                  jax.ShapeDtypeStruct((nq * NCAND // 128, 128), jnp.int32)),
        mesh=mesh,
        compiler_params=pltpu.CompilerParams(needs_layout_passes=False,
                                             use_tc_tiling_on_sc=False),
        scratch_types=[
            pltpu.VMEM((TPW, nq, 128), jnp.float32),
            pltpu.VMEM((CPT,), jnp.int32),
            pltpu.VMEM((CPT, 128), jnp.float32),
            pltpu.SemaphoreType.DMA,
        ],
    )(functools.partial(_sc_topk_body, nq))
    rows, cidx = sc_topk(d2, keys2)

    vals, idx = pl.pallas_call(
        _final_body,
        in_specs=[
            pl.BlockSpec(queries.shape, lambda: (0, 0)),
            pl.BlockSpec((nq * NCAND, 128), lambda: (0, 0)),
            pl.BlockSpec((nq * NCAND // 128, 128), lambda: (0, 0)),
        ],
        out_specs=[
            pl.BlockSpec((nq, TOPK), lambda: (0, 0)),
            pl.BlockSpec((nq, TOPK), lambda: (0, 0)),
        ],
        out_shape=[
            jax.ShapeDtypeStruct((nq, TOPK), jnp.float32),
            jax.ShapeDtypeStruct((nq, TOPK), jnp.int32),
        ],
    )(queries, rows, cidx)
    return vals, idx
